# baseline (speedup 1.0000x reference)
import jax
import jax.numpy as jnp
from jax.experimental import pallas as pl
from jax.experimental.pallas import tpu as pltpu

_E_BLOCK = 16000
_N_BLOCK = 2000


def _edge_kernel(x_ref, w1t_ref, b1t_ref, w2t_ref, b2t_ref, out_ref):
    xt = x_ref[...].astype(jnp.bfloat16).T
    zt = jnp.dot(w1t_ref[...], xt,
                 preferred_element_type=jnp.float32).astype(jnp.bfloat16)
    ht = jnp.maximum(zt + b1t_ref[...], jnp.bfloat16(0.0))
    ot = jnp.dot(w2t_ref[...], ht, preferred_element_type=jnp.float32)
    out_ref[...] = ot + b2t_ref[...]


def _node_kernel(x_ref, w1_ref, b1_ref, w2_ref, b2_ref, o_ref):
    x = x_ref[...].astype(jnp.bfloat16)
    h = jnp.dot(x, w1_ref[...], preferred_element_type=jnp.float32)
    h = jnp.maximum(h + b1_ref[...], 0.0).astype(jnp.bfloat16)
    out = jnp.dot(h, w2_ref[...], preferred_element_type=jnp.float32)
    o_ref[...] = out + b2_ref[...]


def kernel(node_features, edge_features, edge_index, We1, be1, We2, be2,
           Wa1, ba1, Wa2, ba2, Wn1, bn1, Wn2, bn2):
    E, D = edge_features.shape
    N, DN = node_features.shape
    H = We1.shape[1]

    W1t = jnp.concatenate([We1, Wa1], axis=1).T.astype(jnp.bfloat16)
    b1t = jnp.concatenate([be1, ba1])[:, None].astype(jnp.bfloat16)
    W2cat = jnp.zeros((2 * H, 8), jnp.float32)
    W2cat = W2cat.at[:H, 0:1].set(We2)
    W2cat = W2cat.at[H:, 1:5].set(Wa2)
    W2t = W2cat.T.astype(jnp.bfloat16)
    b2t = jnp.zeros((8, 1), jnp.float32)
    b2t = b2t.at[0, 0].set(be2[0])
    b2t = b2t.at[1:5, 0].set(ba2)

    out8 = pl.pallas_call(
        _edge_kernel,
        grid=(E // _E_BLOCK,),
        in_specs=[
            pl.BlockSpec((_E_BLOCK, D), lambda i: (i, 0)),
            pl.BlockSpec((2 * H, D), lambda i: (0, 0)),
            pl.BlockSpec((2 * H, 1), lambda i: (0, 0)),
            pl.BlockSpec((8, 2 * H), lambda i: (0, 0)),
            pl.BlockSpec((8, 1), lambda i: (0, 0)),
        ],
        out_specs=pl.BlockSpec((8, _E_BLOCK), lambda i: (0, i)),
        out_shape=jax.ShapeDtypeStruct((8, E), jnp.float32),
        compiler_params=pltpu.CompilerParams(
            dimension_semantics=("parallel",),
            vmem_limit_bytes=100 * 1024 * 1024),
    )(edge_features, W1t, b1t, W2t, b2t)

    vo = pl.pallas_call(
        _node_kernel,
        grid=(N // _N_BLOCK,),
        in_specs=[
            pl.BlockSpec((_N_BLOCK, DN), lambda i: (i, 0)),
            pl.BlockSpec((DN, H), lambda i: (0, 0)),
            pl.BlockSpec((1, H), lambda i: (0, 0)),
            pl.BlockSpec((H, 2), lambda i: (0, 0)),
            pl.BlockSpec((1, 2), lambda i: (0, 0)),
        ],
        out_specs=pl.BlockSpec((_N_BLOCK, 2), lambda i: (i, 0)),
        out_shape=jax.ShapeDtypeStruct((N, 2), jnp.float32),
        compiler_params=pltpu.CompilerParams(
            dimension_semantics=("parallel",)),
    )(node_features, Wn1.astype(jnp.bfloat16), bn1[None, :],
      Wn2.astype(jnp.bfloat16), bn2[None, :])

    return out8[0, :], out8[1:5, :].T, vo


# speedup vs baseline: 1.0117x; 1.0117x over previous
import jax
import jax.numpy as jnp
from jax.experimental import pallas as pl
from jax.experimental.pallas import tpu as pltpu

_E_BLOCK = 12800
_N_BLOCK = 2000


def _edge_kernel(x_ref, w1t_ref, b1t_ref, w2t_ref, b2t_ref, out_ref):
    xt = x_ref[...].astype(jnp.bfloat16).T
    zt = jnp.dot(w1t_ref[...], xt,
                 preferred_element_type=jnp.float32).astype(jnp.bfloat16)
    ht = jnp.maximum(zt + b1t_ref[...], jnp.bfloat16(0.0))
    ot = jnp.dot(w2t_ref[...], ht, preferred_element_type=jnp.float32)
    out_ref[...] = ot + b2t_ref[...]


def _node_kernel(x_ref, w1_ref, b1_ref, w2_ref, b2_ref, o_ref):
    x = x_ref[...].astype(jnp.bfloat16)
    h = jnp.dot(x, w1_ref[...], preferred_element_type=jnp.float32)
    h = jnp.maximum(h + b1_ref[...], 0.0).astype(jnp.bfloat16)
    out = jnp.dot(h, w2_ref[...], preferred_element_type=jnp.float32)
    o_ref[...] = out + b2_ref[...]


def kernel(node_features, edge_features, edge_index, We1, be1, We2, be2,
           Wa1, ba1, Wa2, ba2, Wn1, bn1, Wn2, bn2):
    E, D = edge_features.shape
    N, DN = node_features.shape
    H = We1.shape[1]

    W1t = jnp.concatenate([We1, Wa1], axis=1).T.astype(jnp.bfloat16)
    b1t = jnp.concatenate([be1, ba1])[:, None].astype(jnp.bfloat16)
    W2cat = jnp.zeros((2 * H, 8), jnp.float32)
    W2cat = W2cat.at[:H, 0:1].set(We2)
    W2cat = W2cat.at[H:, 1:5].set(Wa2)
    W2t = W2cat.T.astype(jnp.bfloat16)
    b2t = jnp.zeros((8, 1), jnp.float32)
    b2t = b2t.at[0, 0].set(be2[0])
    b2t = b2t.at[1:5, 0].set(ba2)

    out8 = pl.pallas_call(
        _edge_kernel,
        grid=(E // _E_BLOCK,),
        in_specs=[
            pl.BlockSpec((_E_BLOCK, D), lambda i: (i, 0)),
            pl.BlockSpec((2 * H, D), lambda i: (0, 0)),
            pl.BlockSpec((2 * H, 1), lambda i: (0, 0)),
            pl.BlockSpec((8, 2 * H), lambda i: (0, 0)),
            pl.BlockSpec((8, 1), lambda i: (0, 0)),
        ],
        out_specs=pl.BlockSpec((8, _E_BLOCK), lambda i: (0, i)),
        out_shape=jax.ShapeDtypeStruct((8, E), jnp.float32),
        compiler_params=pltpu.CompilerParams(
            dimension_semantics=("parallel",)),
    )(edge_features, W1t, b1t, W2t, b2t)

    vo = pl.pallas_call(
        _node_kernel,
        grid=(N // _N_BLOCK,),
        in_specs=[
            pl.BlockSpec((_N_BLOCK, DN), lambda i: (i, 0)),
            pl.BlockSpec((DN, H), lambda i: (0, 0)),
            pl.BlockSpec((1, H), lambda i: (0, 0)),
            pl.BlockSpec((H, 2), lambda i: (0, 0)),
            pl.BlockSpec((1, 2), lambda i: (0, 0)),
        ],
        out_specs=pl.BlockSpec((_N_BLOCK, 2), lambda i: (i, 0)),
        out_shape=jax.ShapeDtypeStruct((N, 2), jnp.float32),
        compiler_params=pltpu.CompilerParams(
            dimension_semantics=("parallel",)),
    )(node_features, Wn1.astype(jnp.bfloat16), bn1[None, :],
      Wn2.astype(jnp.bfloat16), bn2[None, :])

    return out8[0, :], out8[1:5, :].T, vo
